# initial kernel scaffold (unmeasured)
import jax
import jax.numpy as jnp
from jax import lax
from jax.experimental import pallas as pl
from jax.experimental.pallas import tpu as pltpu


def kernel(
    x,
):
    def body(*refs):
        pass

    out_shape = jax.ShapeDtypeStruct(..., jnp.float32)
    return pl.pallas_call(body, out_shape=out_shape)(...)


# baseline (device time: 18016 ns/iter reference)
import jax
import jax.numpy as jnp
from jax import lax
from jax.experimental import pallas as pl
from jax.experimental.pallas import tpu as pltpu

N_DEV = 4


def kernel(x):
    m_per, n = x.shape

    def body(x_ref, out_ref, comm_ref, send_sems, recv_sems):
        my_pos = lax.axis_index("i")
        left = (my_pos - 1) % N_DEV
        right = (my_pos + 1) % N_DEV

        barrier_sem = pltpu.get_barrier_semaphore()
        for nbr in [left, right]:
            pl.semaphore_signal(
                barrier_sem, inc=1,
                device_id=(nbr,), device_id_type=pl.DeviceIdType.MESH,
            )
        pl.semaphore_wait(barrier_sem, 2)

        xv = x_ref[:, :]
        vmax = jnp.max(xv, axis=0, keepdims=True)
        rows = lax.broadcasted_iota(jnp.int32, (m_per, n), 0)
        masked = jnp.where(xv == vmax, rows, jnp.int32(2**30))
        imin = jnp.min(masked, axis=0, keepdims=True)
        gidx = (imin + my_pos * m_per).astype(jnp.float32)

        comm_ref[0, 0:1, :] = vmax
        comm_ref[0, 1:2, :] = gidx

        best_v = vmax
        best_i = gidx

        for h in range(N_DEV - 1):
            rdma = pltpu.make_async_remote_copy(
                src_ref=comm_ref.at[h],
                dst_ref=comm_ref.at[h + 1],
                send_sem=send_sems.at[h],
                recv_sem=recv_sems.at[h],
                device_id=(right,),
                device_id_type=pl.DeviceIdType.MESH,
            )
            rdma.start()
            rdma.wait()

            o_v = comm_ref[h + 1, 0:1, :]
            o_i = comm_ref[h + 1, 1:2, :]
            take = (o_v > best_v) | ((o_v == best_v) & (o_i < best_i))
            best_v = jnp.where(take, o_v, best_v)
            best_i = jnp.where(take, o_i, best_i)

        out_ref[0:1, :] = best_v
        out_ref[1:2, :] = best_i

    return pl.pallas_call(
        body,
        out_shape=jax.ShapeDtypeStruct((2, n), jnp.float32),
        in_specs=[pl.BlockSpec(memory_space=pltpu.VMEM)],
        out_specs=pl.BlockSpec(memory_space=pltpu.VMEM),
        scratch_shapes=[
            pltpu.VMEM((N_DEV, 2, n), jnp.float32),
            pltpu.SemaphoreType.DMA((N_DEV - 1,)),
            pltpu.SemaphoreType.DMA((N_DEV - 1,)),
        ],
        compiler_params=pltpu.CompilerParams(collective_id=0),
    )(x)


# device time: 14519 ns/iter; 1.2409x vs baseline; 1.2409x over previous
import jax
import jax.numpy as jnp
from jax import lax
from jax.experimental import pallas as pl
from jax.experimental.pallas import tpu as pltpu

N_DEV = 4
G = 8


def kernel(x):
    m_per, n = x.shape
    block_m = m_per // G

    def body(x_ref, out_ref, acc_ref, my_ref, comm_ref, send_sems, recv_sems):
        g = pl.program_id(0)
        my_pos = lax.axis_index("i")

        barrier_sem = pltpu.get_barrier_semaphore()

        @pl.when(g == 0)
        def _():
            for d in range(1, N_DEV):
                pl.semaphore_signal(
                    barrier_sem, inc=1,
                    device_id=((my_pos + d) % N_DEV,),
                    device_id_type=pl.DeviceIdType.MESH,
                )

        xv = x_ref[:, :]
        bmax = jnp.max(xv, axis=0, keepdims=True)
        rows = lax.broadcasted_iota(jnp.int32, (block_m, n), 0)
        masked = jnp.where(xv == bmax, rows, jnp.int32(2**30))
        bimin = jnp.min(masked, axis=0, keepdims=True)
        bidx = (bimin + g * block_m + my_pos * m_per).astype(jnp.float32)

        @pl.when(g == 0)
        def _():
            acc_ref[0:1, :] = bmax
            acc_ref[1:2, :] = bidx

        @pl.when(g > 0)
        def _():
            best_v = acc_ref[0:1, :]
            take = bmax > best_v
            acc_ref[0:1, :] = jnp.where(take, bmax, best_v)
            acc_ref[1:2, :] = jnp.where(take, bidx, acc_ref[1:2, :])

        @pl.when(g == G - 1)
        def _():
            my_ref[:, :] = acc_ref[:, :]
            pl.semaphore_wait(barrier_sem, N_DEV - 1)

            rdmas = []
            for d in range(1, N_DEV):
                rdma = pltpu.make_async_remote_copy(
                    src_ref=my_ref,
                    dst_ref=comm_ref.at[N_DEV - d],
                    send_sem=send_sems.at[d - 1],
                    recv_sem=recv_sems.at[N_DEV - d],
                    device_id=((my_pos + d) % N_DEV,),
                    device_id_type=pl.DeviceIdType.MESH,
                )
                rdma.start()
                rdmas.append(rdma)

            best_v = acc_ref[0:1, :]
            best_i = acc_ref[1:2, :]
            for d in (1, 3, 2):
                rdmas[d - 1].wait_recv()
                o_v = comm_ref[N_DEV - d, 0:1, :]
                o_i = comm_ref[N_DEV - d, 1:2, :]
                take = (o_v > best_v) | ((o_v == best_v) & (o_i < best_i))
                best_v = jnp.where(take, o_v, best_v)
                best_i = jnp.where(take, o_i, best_i)

            out_ref[0:1, :] = best_v
            out_ref[1:2, :] = best_i

            for rdma in rdmas:
                rdma.wait_send()

    return pl.pallas_call(
        body,
        grid=(G,),
        out_shape=jax.ShapeDtypeStruct((2, n), jnp.float32),
        in_specs=[
            pl.BlockSpec((block_m, n), lambda g: (g, 0), memory_space=pltpu.VMEM)
        ],
        out_specs=pl.BlockSpec((2, n), lambda g: (0, 0), memory_space=pltpu.VMEM),
        scratch_shapes=[
            pltpu.VMEM((2, n), jnp.float32),
            pltpu.VMEM((2, n), jnp.float32),
            pltpu.VMEM((N_DEV, 2, n), jnp.float32),
            pltpu.SemaphoreType.DMA((N_DEV - 1,)),
            pltpu.SemaphoreType.DMA((N_DEV,)),
        ],
        compiler_params=pltpu.CompilerParams(collective_id=0),
    )(x)


# device time: 14260 ns/iter; 1.2634x vs baseline; 1.0182x over previous
import jax
import jax.numpy as jnp
from jax import lax
from jax.experimental import pallas as pl
from jax.experimental.pallas import tpu as pltpu

N_DEV = 4


def kernel(x):
    m_per, n = x.shape

    def body(x_ref, out_ref, my_ref, comm_ref, send_sems, recv_sems):
        my_pos = lax.axis_index("i")

        barrier_sem = pltpu.get_barrier_semaphore()
        for d in range(1, N_DEV):
            pl.semaphore_signal(
                barrier_sem, inc=1,
                device_id=((my_pos + d) % N_DEV,),
                device_id_type=pl.DeviceIdType.MESH,
            )

        xv = x_ref[:, :]
        vmax = jnp.max(xv, axis=0, keepdims=True)
        rows = lax.broadcasted_iota(jnp.int32, (m_per, n), 0)
        masked = jnp.where(xv == vmax, rows, jnp.int32(2**30))
        imin = jnp.min(masked, axis=0, keepdims=True)
        gidx = (imin + my_pos * m_per).astype(jnp.float32)

        my_ref[0:1, :] = vmax
        my_ref[1:2, :] = gidx

        pl.semaphore_wait(barrier_sem, N_DEV - 1)

        rdmas = []
        for d in range(1, N_DEV):
            rdma = pltpu.make_async_remote_copy(
                src_ref=my_ref,
                dst_ref=comm_ref.at[N_DEV - d],
                send_sem=send_sems.at[d - 1],
                recv_sem=recv_sems.at[N_DEV - d],
                device_id=((my_pos + d) % N_DEV,),
                device_id_type=pl.DeviceIdType.MESH,
            )
            rdma.start()
            rdmas.append(rdma)

        best_v = my_ref[0:1, :]
        best_i = my_ref[1:2, :]
        for d in (1, 3, 2):
            rdmas[d - 1].wait_recv()
            o_v = comm_ref[N_DEV - d, 0:1, :]
            o_i = comm_ref[N_DEV - d, 1:2, :]
            take = (o_v > best_v) | ((o_v == best_v) & (o_i < best_i))
            best_v = jnp.where(take, o_v, best_v)
            best_i = jnp.where(take, o_i, best_i)

        out_ref[0:1, :] = best_v
        out_ref[1:2, :] = best_i

        for rdma in rdmas:
            rdma.wait_send()

    return pl.pallas_call(
        body,
        out_shape=jax.ShapeDtypeStruct((2, n), jnp.float32),
        in_specs=[pl.BlockSpec(memory_space=pltpu.VMEM)],
        out_specs=pl.BlockSpec(memory_space=pltpu.VMEM),
        scratch_shapes=[
            pltpu.VMEM((2, n), jnp.float32),
            pltpu.VMEM((N_DEV, 2, n), jnp.float32),
            pltpu.SemaphoreType.DMA((N_DEV - 1,)),
            pltpu.SemaphoreType.DMA((N_DEV,)),
        ],
        compiler_params=pltpu.CompilerParams(collective_id=0),
    )(x)
